# baseline (speedup 1.0000x reference)
import functools

import jax
import jax.numpy as jnp
from jax import lax
from jax.experimental import pallas as pl
from jax.experimental.pallas import tpu as pltpu
from jax.experimental.pallas import tpu_sc as plsc

_CHUNK = 128
_LANES = 16


@functools.lru_cache(maxsize=None)
def _build(V, D, B):
    info = plsc.get_sparse_core_info()
    nw = info.num_cores * info.num_subcores
    pack = 128 // D
    assert B % (8 * nw) == 0 and V % pack == 0
    b_per_w = B // nw
    ch = min(_CHUNK, b_per_w)
    n_ch = b_per_w // ch
    assert n_ch * ch == b_per_w
    out_rows_w = b_per_w // pack

    mesh = plsc.VectorSubcoreMesh(core_axis_name="c", subcore_axis_name="s")

    @functools.partial(
        pl.kernel,
        mesh=mesh,
        out_type=jax.ShapeDtypeStruct((B // pack, 128), jnp.float32),
        compiler_params=pltpu.CompilerParams(needs_layout_passes=False),
        scratch_types=[
            pltpu.VMEM((n_ch, ch), jnp.int32),
            pltpu.VMEM((n_ch, ch), jnp.int32),
            pltpu.VMEM((b_per_w, 128), jnp.float32),
            pltpu.VMEM((out_rows_w, 128), jnp.float32),
            pltpu.SemaphoreType.DMA,
        ],
    )
    def gather_kernel(table_hbm, idx_hbm, out_hbm, idx_v, hi_v, rows_v,
                      out_v, sem):
        wid = lax.axis_index("s") * info.num_cores + lax.axis_index("c")
        base = wid * b_per_w
        for c in range(n_ch):
            pltpu.sync_copy(idx_hbm.at[pl.ds(base + c * ch, ch)], idx_v.at[c])
        for c in range(n_ch):
            for k in range(ch // _LANES):
                sl = pl.ds(k * _LANES, _LANES)
                hi_v[c, sl] = lax.shift_right_logical(idx_v[c, sl], 3)
        copies = [
            pltpu.async_copy(
                table_hbm.at[hi_v.at[c]],
                rows_v.at[pl.ds(c * ch, ch)],
                sem,
            )
            for c in range(n_ch)
        ]
        for cp in copies:
            cp.wait()

        iota = lax.iota(jnp.int32, _LANES)

        def body(g, _):
            b0 = g * _LANES
            c = b0 // ch
            vidx = idx_v[c, pl.ds(b0 - c * ch, _LANES)]
            lo = (vidx & 7) * D
            rowids = b0 + iota
            orow = lax.shift_right_logical(rowids, 3)
            ocol0 = (rowids & 7) * D
            for j in range(D):
                val = plsc.load_gather(rows_v, [rowids, lo + j])
                plsc.store_scatter(out_v, [orow, ocol0 + j], val)
            return _

        for g in range(b_per_w // _LANES):
            body(g, None)

        pltpu.sync_copy(out_v, out_hbm.at[pl.ds(wid * out_rows_w, out_rows_w)])

    def run(table, idx):
        t2 = table.reshape(V // pack, 128)
        out = gather_kernel(t2, idx)
        return out.reshape(B, D)

    return run


def kernel(subject_embeddings, subject_indices):
    V, D = subject_embeddings.shape
    (B,) = subject_indices.shape
    idx = subject_indices.astype(jnp.int32)
    return _build(V, D, B)(subject_embeddings, idx)


# speedup vs baseline: 6.1054x; 6.1054x over previous
import functools

import jax
import jax.numpy as jnp
from jax import lax
from jax.experimental import pallas as pl
from jax.experimental.pallas import tpu as pltpu
from jax.experimental.pallas import tpu_sc as plsc

_WAVE = 16


@functools.lru_cache(maxsize=None)
def _build(V, D, B):
    info = plsc.get_sparse_core_info()
    nw = info.num_cores * info.num_subcores
    assert B % (8 * nw) == 0
    b_per_w = B // nw
    n_waves = b_per_w // _WAVE
    assert n_waves % 2 == 0

    mesh = plsc.VectorSubcoreMesh(core_axis_name="c", subcore_axis_name="s")

    @functools.partial(
        pl.kernel,
        mesh=mesh,
        out_type=jax.ShapeDtypeStruct((D, B), jnp.float32),
        compiler_params=pltpu.CompilerParams(needs_layout_passes=False),
        scratch_types=[
            pltpu.VMEM((b_per_w,), jnp.int32),
            pltpu.VMEM((_WAVE, D, 128), jnp.float32),
            pltpu.VMEM((_WAVE, D, 128), jnp.float32),
            pltpu.VMEM((D, b_per_w), jnp.float32),
            pltpu.SemaphoreType.DMA,
            pltpu.SemaphoreType.DMA,
        ],
    )
    def gather_kernel(table_hbm, idx_hbm, out_hbm, idx_v, buf0, buf1,
                      land_v, sem0, sem1):
        wid = lax.axis_index("s") * info.num_cores + lax.axis_index("c")
        base = wid * b_per_w
        pltpu.sync_copy(idx_hbm.at[pl.ds(base, b_per_w)], idx_v)
        iota = lax.iota(jnp.int32, 16)

        def fire(w, buf, sem):
            voff = idx_v[pl.ds(w * _WAVE, _WAVE)] & ~jnp.int32(127)
            for k in range(_WAVE):
                off = pl.multiple_of(voff[k], 128)
                pltpu.async_copy(
                    table_hbm.at[pl.ds(0, D), pl.ds(off, 128)],
                    buf.at[k], sem,
                )

        def select(w, buf, sem):
            for k in range(_WAVE):
                pltpu.make_async_copy(
                    table_hbm.at[pl.ds(0, D), pl.ds(0, 128)], buf.at[k], sem
                ).wait()
            vlane = idx_v[pl.ds(w * _WAVE, _WAVE)] & jnp.int32(127)
            for k in range(_WAVE):
                lane = jnp.full((16,), vlane[k], jnp.int32)
                col = jnp.full((16,), w * _WAVE + k, jnp.int32)
                val = plsc.load_gather(buf.at[k], [iota, lane])
                plsc.store_scatter(land_v, [iota, col], val)

        fire(0, buf0, sem0)
        fire(1, buf1, sem1)

        def pair(p):
            w0 = p * 2
            select(w0, buf0, sem0)

            @pl.when(w0 + 2 < n_waves)
            def _():
                fire(w0 + 2, buf0, sem0)

            select(w0 + 1, buf1, sem1)

            @pl.when(w0 + 3 < n_waves)
            def _():
                fire(w0 + 3, buf1, sem1)

        pl.loop(0, n_waves // 2)(pair)
        pltpu.sync_copy(land_v, out_hbm.at[pl.ds(0, D), pl.ds(base, b_per_w)])

    def run(table, idx):
        out_t = gather_kernel(table.T, idx)
        return out_t.T

    return run


def kernel(subject_embeddings, subject_indices):
    V, D = subject_embeddings.shape
    (B,) = subject_indices.shape
    idx = subject_indices.astype(jnp.int32)
    return _build(V, D, B)(subject_embeddings, idx)
